# baseline (speedup 1.0000x reference)
import functools

import jax
import jax.numpy as jnp
from jax import lax
from jax.experimental import pallas as pl
from jax.experimental.pallas import tpu as pltpu
from jax.experimental.pallas import tpu_sc as plsc

_NC = 2
_NS = 16
_NW = _NC * _NS
_NA = 6
_NB = 4
_LOOKAHEAD = 4


@functools.partial(jax.jit, static_argnames=("b", "tp"))
def _gather_sc(idx_flat, table_3, b, tp):
    ng = tp // 8
    b_per_w = b // _NW
    n_units = b_per_w * ng
    dp = 8 * 128
    mesh = plsc.VectorSubcoreMesh(core_axis_name="c", subcore_axis_name="s")

    @functools.partial(
        pl.kernel,
        out_type=jax.ShapeDtypeStruct((b, tp, dp), jnp.float32),
        mesh=mesh,
        scratch_types=[
            pltpu.VMEM((b_per_w * tp,), jnp.int32),
            pltpu.VMEM((_NA, 8, 8, 128), jnp.float32),
            pltpu.VMEM((_NB, 8, dp), jnp.float32),
            pltpu.SemaphoreType.DMA,
            pltpu.SemaphoreType.DMA,
        ],
    )
    def k(idx_hbm, table_hbm, out_hbm, idx_v, abuf, bbuf, gsem, wsem):
        wid = lax.axis_index("s") * _NC + lax.axis_index("c")
        ibase = wid * b_per_w * tp
        bbase = wid * b_per_w
        pltpu.sync_copy(idx_hbm.at[pl.ds(ibase, b_per_w * tp)], idx_v)

        def start_gather(u):
            pltpu.make_async_copy(
                table_hbm.at[idx_v.at[pl.ds(u * 8, 8)]],
                abuf.at[lax.rem(u, _NA)],
                gsem,
            ).start()

        for u in range(_LOOKAHEAD):
            start_gather(u)

        @pl.loop(0, n_units)
        def _unit(u):
            ga = lax.rem(u, _NA)
            gb = lax.rem(u, _NB)
            @pl.when(u >= _NB)
            def _():
                pltpu.make_async_copy(bbuf.at[gb], out_hbm.at[0, pl.ds(0, 8)], wsem).wait()
            pltpu.make_async_copy(
                table_hbm.at[idx_v.at[pl.ds(u * 8, 8)]], abuf.at[ga], gsem
            ).wait()
            for r in range(8):
                for j in range(8):
                    for kk in range(8):
                        bbuf[gb, r, pl.ds(j * 128 + kk * 16, 16)] = (
                            abuf[ga, r, j, pl.ds(kk * 16, 16)]
                        )
            bj = lax.div(u, ng)
            gi = lax.rem(u, ng)
            pltpu.make_async_copy(
                bbuf.at[gb],
                out_hbm.at[bbase + bj, pl.ds(gi * 8, 8)],
                wsem,
            ).start()
            @pl.when(u + _LOOKAHEAD < n_units)
            def _():
                start_gather(u + _LOOKAHEAD)

        @pl.loop(0, min(_NB, n_units))
        def _drain(u):
            pltpu.make_async_copy(
                bbuf.at[0], out_hbm.at[0, pl.ds(0, 8)], wsem
            ).wait()

    return k(idx_flat, table_3)


def kernel(idx, table):
    b, t = idx.shape
    v, d = table.shape
    tpad = (t + 7) // 8 * 8
    dpad = (d + 127) // 128 * 128
    idx_p = jnp.pad(idx.astype(jnp.int32), ((0, 0), (0, tpad - t)), mode="wrap")
    table_3 = jnp.pad(table, ((0, 0), (0, dpad - d))).reshape(v, 8, 128)
    out = _gather_sc(idx_p.reshape(-1), table_3, b, tpad)
    return out[:, :t, :d]


# speedup vs baseline: 1.6566x; 1.6566x over previous
import functools

import jax
import jax.numpy as jnp
from jax import lax
from jax.experimental import pallas as pl
from jax.experimental.pallas import tpu as pltpu
from jax.experimental.pallas import tpu_sc as plsc

_NC = 2
_NS = 16
_NW = _NC * _NS
_NA = 6
_NB = 4
_LOOKAHEAD = 4


@functools.partial(jax.jit, static_argnames=("b", "tp"))
def _gather_sc(idx_flat, table_3, b, tp):
    ng = tp // 8
    b_per_w = b // _NW
    n_units = b_per_w * ng
    dp = 8 * 128
    mesh = plsc.VectorSubcoreMesh(core_axis_name="c", subcore_axis_name="s")

    @functools.partial(
        pl.kernel,
        out_type=jax.ShapeDtypeStruct((b, tp, dp), jnp.float32),
        mesh=mesh,
        scratch_types=[
            pltpu.VMEM((b_per_w * tp,), jnp.int32),
            pltpu.VMEM((_NA, 8, 8, 128), jnp.float32),
            pltpu.VMEM((_NB, 8, dp), jnp.float32),
            pltpu.SemaphoreType.DMA,
            pltpu.SemaphoreType.DMA,
        ],
    )
    def k(idx_hbm, table_hbm, out_hbm, idx_v, abuf, bbuf, gsem, wsem):
        wid = lax.axis_index("s") * _NC + lax.axis_index("c")
        ibase = wid * b_per_w * tp
        bbase = wid * b_per_w
        pltpu.sync_copy(idx_hbm.at[pl.ds(ibase, b_per_w * tp)], idx_v)

        def start_gather(u):
            pltpu.make_async_copy(
                table_hbm.at[idx_v.at[pl.ds(u * 8, 8)]],
                abuf.at[lax.rem(u, _NA)],
                gsem,
            ).start()

        for u in range(_LOOKAHEAD):
            start_gather(u)

        @pl.loop(0, n_units)
        def _unit(u):
            ga = lax.rem(u, _NA)
            gb = lax.rem(u, _NB)
            @pl.when(u >= _NB)
            def _():
                pltpu.make_async_copy(bbuf.at[gb], out_hbm.at[0, pl.ds(0, 8)], wsem).wait()
            pltpu.make_async_copy(
                table_hbm.at[idx_v.at[pl.ds(u * 8, 8)]], abuf.at[ga], gsem
            ).wait()
            @plsc.parallel_loop(0, 8, unroll=2)
            def _row(r):
                for j in range(8):
                    vals = [
                        abuf[ga, r, j, pl.ds(kk * 16, 16)] for kk in range(8)
                    ]
                    for kk in range(8):
                        bbuf[gb, r, pl.ds(j * 128 + kk * 16, 16)] = vals[kk]
            bj = lax.div(u, ng)
            gi = lax.rem(u, ng)
            pltpu.make_async_copy(
                bbuf.at[gb],
                out_hbm.at[bbase + bj, pl.ds(gi * 8, 8)],
                wsem,
            ).start()
            @pl.when(u + _LOOKAHEAD < n_units)
            def _():
                start_gather(u + _LOOKAHEAD)

        @pl.loop(0, min(_NB, n_units))
        def _drain(u):
            pltpu.make_async_copy(
                bbuf.at[0], out_hbm.at[0, pl.ds(0, 8)], wsem
            ).wait()

    return k(idx_flat, table_3)


def kernel(idx, table):
    b, t = idx.shape
    v, d = table.shape
    tpad = (t + 7) // 8 * 8
    dpad = (d + 127) // 128 * 128
    idx_p = jnp.pad(idx.astype(jnp.int32), ((0, 0), (0, tpad - t)), mode="wrap")
    table_3 = jnp.pad(table, ((0, 0), (0, dpad - d))).reshape(v, 8, 128)
    out = _gather_sc(idx_p.reshape(-1), table_3, b, tpad)
    return out[:, :t, :d]
